# Initial kernel scaffold; baseline (speedup 1.0000x reference)
import jax
import jax.numpy as jnp
from jax.experimental import pallas as pl


def kernel(y, incidence, dxdt, W, b):
    raise NotImplementedError("write your pallas kernel here")


# speedup vs baseline: 4.4572x; 4.4572x over previous
import dataclasses
import functools

import jax
import jax.numpy as jnp
from jax import lax
from jax.experimental import pallas as pl
from jax.experimental.pallas import tpu as pltpu
from jax.experimental.pallas import tpu_sc as plsc

NC = 2
NS = 16
L = 16
NW = NC * NS

SP = 10240
K = 128
ZR = 64


def _sc_gather_segsum(table, gidx, sidx):
  T, D = table.shape
  NNZp = gidx.shape[0]
  per_w = NNZp // NW
  CH = per_w // K
  stripe = SP // NS

  mesh = plsc.VectorSubcoreMesh(
      core_axis_name="c", subcore_axis_name="s", num_cores=NC, num_subcores=NS
  )

  cp = pltpu.CompilerParams()
  if "needs_layout_passes" in pltpu.CompilerParams.__dataclass_fields__:
    cp = dataclasses.replace(cp, needs_layout_passes=False)

  @functools.partial(
      pl.kernel,
      compiler_params=cp,
      out_type=(
          jax.ShapeDtypeStruct((NC, SP, D), jnp.float32),
          jax.ShapeDtypeStruct((NW, SP), jnp.float32),
      ),
      mesh=mesh,
      scratch_types=[
          pltpu.VMEM((K,), jnp.int32),
          pltpu.VMEM((K,), jnp.int32),
          pltpu.VMEM((K, D), jnp.float32),
          pltpu.VMEM((SP,), jnp.float32),
          pltpu.VMEM((ZR, D), jnp.float32),
          pltpu.VMEM_SHARED((SP, D), jnp.float32),
          pltpu.SemaphoreType.DMA,
      ],
  )
  def k(table_h, gidx_h, sidx_h, acc_h, cnt_h,
        gidx_v, sidx_v, rows_v, cnt_v, zb, acc_s, sem):
    c = lax.axis_index("c")
    s = lax.axis_index("s")
    wid = s * NC + c

    zero16 = jnp.zeros((L,), jnp.float32)
    ones16 = jnp.ones((L,), jnp.float32)

    @pl.loop(0, ZR)
    def _(i):
      @pl.loop(0, D // L)
      def _(j):
        zb[i, pl.ds(j * L, L)] = zero16

    @pl.loop(0, SP // L)
    def _(i):
      cnt_v[pl.ds(i * L, L)] = zero16

    row0 = s * stripe

    @pl.loop(0, stripe // ZR)
    def _(j):
      pltpu.sync_copy(zb, acc_s.at[pl.ds(row0 + j * ZR, ZR)])

    plsc.subcore_barrier()

    base_w = wid * per_w

    @pl.loop(0, CH)
    def _(i):
      base = base_w + i * K
      pltpu.sync_copy(gidx_h.at[pl.ds(base, K)], gidx_v)
      pltpu.sync_copy(sidx_h.at[pl.ds(base, K)], sidx_v)
      pltpu.async_copy(table_h.at[gidx_v], rows_v, sem).wait()
      pltpu.sync_copy(rows_v, acc_s.at[sidx_v], add=True)

      @pl.loop(0, K // L)
      def _(g):
        idx16 = sidx_v[pl.ds(g * L, L)]
        plsc.addupdate_scatter(cnt_v, [idx16], ones16)

    plsc.subcore_barrier()

    pltpu.sync_copy(acc_s.at[pl.ds(row0, stripe)],
                    acc_h.at[c].at[pl.ds(row0, stripe)])
    pltpu.sync_copy(cnt_v, cnt_h.at[wid])

  return k(table, gidx, sidx)


def _tc_norm(acc, cnt):
  _, sp, d = acc.shape
  B = 512

  def body(a_ref, c_ref, o_ref):
    n = jnp.sum(c_ref[...], axis=0)
    inv = 1.0 / jnp.maximum(n, 1.0)
    o_ref[...] = (a_ref[0] + a_ref[1]) * inv[:, None]

  return pl.pallas_call(
      body,
      grid=(sp // B,),
      in_specs=[
          pl.BlockSpec((NC, B, d), lambda i: (0, i, 0)),
          pl.BlockSpec((NW, B), lambda i: (0, i)),
      ],
      out_specs=pl.BlockSpec((B, d), lambda i: (i, 0)),
      out_shape=jax.ShapeDtypeStruct((sp, d), jnp.float32),
  )(acc, cnt)


def _tc_final(acc, cnt, dxdt_p, wc, bc):
  _, sp, d = acc.shape
  cdim = wc.shape[0]
  B = 512

  def body(a_ref, c_ref, dx_ref, w_ref, b_ref, o_ref):
    n = jnp.sum(c_ref[...], axis=0)
    inv = 1.0 / jnp.maximum(n, 1.0)
    agg = (a_ref[0] + a_ref[1]) * inv[:, None]
    out = jnp.zeros((B, d), jnp.float32)
    for cc in range(cdim):
      raw = lax.dot_general(
          agg, w_ref[cc], (((1,), (0,)), ((), ())),
          preferred_element_type=jnp.float32,
          precision=lax.Precision.HIGHEST,
      )
      raw = jnp.maximum(raw + b_ref[cc][None, :], 0.0)
      out = out + raw * dx_ref[:, cc][:, None]
    o_ref[...] = out

  return pl.pallas_call(
      body,
      grid=(sp // B,),
      in_specs=[
          pl.BlockSpec((NC, B, d), lambda i: (0, i, 0)),
          pl.BlockSpec((NW, B), lambda i: (0, i)),
          pl.BlockSpec((B, cdim), lambda i: (i, 0)),
          pl.BlockSpec((cdim, d, d), lambda i: (0, 0, 0)),
          pl.BlockSpec((cdim, d), lambda i: (0, 0)),
      ],
      out_specs=pl.BlockSpec((B, d), lambda i: (i, 0)),
      out_shape=jax.ShapeDtypeStruct((sp, d), jnp.float32),
  )(acc, cnt, dxdt_p, wc, bc)


def kernel(y, incidence, dxdt, W, b):
  n, d = y.shape
  cdim = dxdt.shape[1]
  nnz = incidence.shape[1]

  node_idx = incidence[0]
  edge_idx = incidence[1]

  chunk = NW * K
  nnzp = ((nnz + chunk - 1) // chunk) * chunk
  pad = nnzp - nnz
  gpad = jnp.zeros((pad,), jnp.int32)
  spad = jnp.full((pad,), SP - 1, jnp.int32)

  acc1, cnt1 = _sc_gather_segsum(
      y,
      jnp.concatenate([node_idx, gpad]),
      jnp.concatenate([edge_idx, spad]),
  )
  edge_feat = _tc_norm(acc1, cnt1)

  acc2, cnt2 = _sc_gather_segsum(
      edge_feat,
      jnp.concatenate([edge_idx, gpad]),
      jnp.concatenate([node_idx, spad]),
  )

  wc = jnp.transpose(W.reshape(d, d, cdim), (2, 0, 1))
  bc = jnp.transpose(b.reshape(d, cdim), (1, 0))
  dxdt_p = jnp.concatenate(
      [dxdt, jnp.zeros((SP - n, cdim), jnp.float32)], axis=0
  )

  drift = _tc_final(acc2, cnt2, dxdt_p, wc, bc)
  return drift[:n]
